# baseline (speedup 1.0000x reference)
import functools

import jax
import jax.numpy as jnp
from jax import lax
from jax.experimental import pallas as pl
from jax.experimental.pallas import tpu as pltpu
from jax.experimental.pallas import tpu_sc as plsc

N = 10000
E = 320000
D = 128
ED = 16
B = 64
NCORE = 2
NSUB = 16
NW = NCORE * NSUB
C = 64
NCHUNK = E // C
NCHUNK_P = 5120
TPC = NCHUNK_P // NW
SB = 4
NSB = TPC // SB
NPAD = 10112
ROWS_PER_SUB = NPAD // NSUB


def _eterm_body(eat_ref, we1_ref, ae1_ref, we2_ref, ae2_ref, o1_ref, o2_ref):
    w1 = jnp.dot(we1_ref[...], ae1_ref[...], preferred_element_type=jnp.float32)
    w2 = jnp.dot(we2_ref[...], ae2_ref[...], preferred_element_type=jnp.float32)
    a = eat_ref[...]
    m1 = lax.dot_general(
        w1, a, (((0,), (0,)), ((), ())), preferred_element_type=jnp.float32
    )
    m2 = lax.dot_general(
        w2, a, (((0,), (0,)), ((), ())), preferred_element_type=jnp.float32
    )
    blk = m1.shape[1]
    gidx = lax.broadcasted_iota(jnp.int32, (1, blk), 1) + pl.program_id(0) * blk
    m1 = jnp.where(gidx < E, m1, -1e30)
    m2 = jnp.where(gidx < E, m2, -1e30)
    o1_ref[...] = m1[0]
    o2_ref[...] = m2[0]


def _eterms(edge_attr_t, We1, ae1, We2, ae2):
    blk = 32768
    EP = NCHUNK_P * C
    grid = EP // blk
    out = pl.pallas_call(
        _eterm_body,
        grid=(grid,),
        in_specs=[
            pl.BlockSpec((ED, blk), lambda i: (0, i)),
            pl.BlockSpec((ED, D), lambda i: (0, 0)),
            pl.BlockSpec((D, 1), lambda i: (0, 0)),
            pl.BlockSpec((ED, D), lambda i: (0, 0)),
            pl.BlockSpec((D, 1), lambda i: (0, 0)),
        ],
        out_specs=[
            pl.BlockSpec((blk,), lambda i: (i,)),
            pl.BlockSpec((blk,), lambda i: (i,)),
        ],
        out_shape=[
            jax.ShapeDtypeStruct((EP,), jnp.float32),
            jax.ShapeDtypeStruct((EP,), jnp.float32),
        ],
    )(edge_attr_t, We1, ae1.reshape(D, 1), We2, ae2.reshape(D, 1))
    return out[0].reshape(NCHUNK_P, C), out[1].reshape(NCHUNK_P, C)


def _prep_body(x_ref, w_ref, asrc_ref, adst_ref, h_ref, s_ref, d_ref):
    h = jnp.dot(x_ref[...], w_ref[...], preferred_element_type=jnp.float32)
    h_ref[...] = h
    s_ref[...] = jnp.sum(h * asrc_ref[...], axis=1)
    d_ref[...] = jnp.sum(h * adst_ref[...], axis=1)


def _prep(xin, W, asrc, adst):
    return pl.pallas_call(
        _prep_body,
        out_shape=[
            jax.ShapeDtypeStruct((N, D), jnp.float32),
            jax.ShapeDtypeStruct((N,), jnp.float32),
            jax.ShapeDtypeStruct((N,), jnp.float32),
        ],
    )(xin, W, asrc.reshape(1, D), adst.reshape(1, D))


def _layer_norm(v, g, b):
    mu = jnp.mean(v, axis=-1, keepdims=True)
    var = jnp.mean((v - mu) ** 2, axis=-1, keepdims=True)
    return (v - mu) / jnp.sqrt(var + 1e-5) * g + b


def _combine(acc_ref, denp_ref, b_ref):
    asum = acc_ref[0] + acc_ref[1]
    u = asum[: N, :]
    ones = jnp.ones((NW, 1), jnp.float32)
    den = lax.dot_general(
        denp_ref[...], ones, (((0,), (0,)), ((), ())),
        preferred_element_type=jnp.float32,
    )
    return u / (den[: N] + 1e-16) + b_ref[...]


def _mid_body(acc_ref, denp_ref, b_ref, g_ref, be_ref, w_ref, asrc_ref,
              adst_ref, h_ref, s_ref, d_ref):
    x1 = _combine(acc_ref, denp_ref, b_ref)
    x2 = _layer_norm(x1, g_ref[...], be_ref[...])
    h = jnp.dot(x2, w_ref[...], preferred_element_type=jnp.float32)
    h_ref[...] = h
    s_ref[...] = jnp.sum(h * asrc_ref[...], axis=1)
    d_ref[...] = jnp.sum(h * adst_ref[...], axis=1)


def _mid(acc, denp, b1, g1, be1, W2, asrc2, adst2):
    return pl.pallas_call(
        _mid_body,
        out_shape=[
            jax.ShapeDtypeStruct((N, D), jnp.float32),
            jax.ShapeDtypeStruct((N,), jnp.float32),
            jax.ShapeDtypeStruct((N,), jnp.float32),
        ],
    )(acc, denp, b1.reshape(1, D), g1.reshape(1, D), be1.reshape(1, D), W2,
      asrc2.reshape(1, D), adst2.reshape(1, D))


def _final_body(acc_ref, denp_ref, b_ref, g_ref, be_ref, batch_ref, wl_ref,
                bl_ref, gl_ref, bel_ref, out_ref):
    x3 = _combine(acc_ref, denp_ref, b_ref)
    x4 = _layer_norm(x3, g_ref[...], be_ref[...])
    oh = jnp.where(
        batch_ref[...] == lax.broadcasted_iota(jnp.int32, (N, B), 1), 1.0, 0.0
    )
    sums = lax.dot_general(
        oh, x4, (((0,), (0,)), ((), ())), preferred_element_type=jnp.float32
    )
    counts = lax.dot_general(
        oh, jnp.ones((N, 1), jnp.float32), (((0,), (0,)), ((), ())),
        preferred_element_type=jnp.float32,
    )
    pooled = sums / jnp.maximum(counts, 1.0)
    x5 = jnp.dot(pooled, wl_ref[...], preferred_element_type=jnp.float32)
    x5 = x5 + bl_ref[...]
    x6 = _layer_norm(x5, gl_ref[...], bel_ref[...])
    out_ref[...] = jnp.maximum(x6, 0.0)


def _final(acc, denp, b2, g2, be2, batch2d, Wl, bl, gl, bel):
    return pl.pallas_call(
        _final_body,
        out_shape=jax.ShapeDtypeStruct((B, D), jnp.float32),
    )(acc, denp, b2.reshape(1, D), g2.reshape(1, D), be2.reshape(1, D),
      batch2d, Wl, bl.reshape(1, D), gl.reshape(1, D), bel.reshape(1, D))


def _sc_edge_body(h_hbm, s_hbm, d_hbm, pk_hbm, dst_hbm,
                  out_hbm, den_hbm,
                  s_v, d_v, pk_v, dstb_v, ea_v, rows_v, den_v,
                  k16_v, c16_v, v16_v, acc_sh, sem0, sem1, isem0, isem1,
                  ssem0, ssem1):
    cid = lax.axis_index("c")
    sid = lax.axis_index("s")
    wid = sid * NCORE + cid
    ii = lax.iota(jnp.int32, 16)
    sems = (sem0, sem1)
    isems = (isem0, isem1)
    ssems = (ssem0, ssem1)

    def _zrow(i, _):
        for k in range(D // 16):
            rows_v[0, i, pl.ds(k * 16, 16)] = jnp.zeros((16,), jnp.float32)
        return 0

    lax.fori_loop(0, C, _zrow, 0)

    def _zden(i, _):
        den_v[pl.ds(i * 16, 16)] = jnp.zeros((16,), jnp.float32)
        return 0

    lax.fori_loop(0, N // 16, _zden, 0)
    for j in range(ROWS_PER_SUB // C):
        pltpu.sync_copy(
            rows_v.at[0],
            acc_sh.at[pl.ds(sid * ROWS_PER_SUB + j * C, C)],
        )
    _rem = ROWS_PER_SUB % C
    if _rem:
        pltpu.sync_copy(
            rows_v.at[0, pl.ds(0, _rem)],
            acc_sh.at[pl.ds(sid * ROWS_PER_SUB + (ROWS_PER_SUB // C) * C,
                            _rem)],
        )

    pltpu.sync_copy(s_hbm, s_v)
    pltpu.sync_copy(d_hbm, d_v)
    plsc.subcore_barrier()

    def _compute(jj, bb, pb):
        @plsc.parallel_loop(0, C // 16, unroll=C // 16)
        def _alpha(g):
            si = pk_v[pb, jj, pl.ds(g * 16, 16)]
            di = dstb_v[pb, jj, pl.ds(g * 16, 16)]
            et = plsc.bitcast(pk_v[pb, jj, pl.ds(C + g * 16, 16)],
                              jnp.float32)
            a = plsc.load_gather(s_v, [si]) + plsc.load_gather(d_v, [di]) + et
            a = jnp.maximum(a, 0.2 * a)
            ea = jnp.exp(a)
            ea_v[pl.ds(g * 16, 16)] = ea

        @plsc.parallel_loop(0, C, unroll=6)
        def _scale(e):
            bc = plsc.load_gather(ea_v, [jnp.full((16,), e, jnp.int32)])
            for k in range(D // 16):
                rows_v[bb, e, pl.ds(k * 16, 16)] = (
                    rows_v[bb, e, pl.ds(k * 16, 16)] * bc
                )

    def _denom(jj, pb):
        for g in range(C // 16):
            di = dstb_v[pb, jj, pl.ds(g * 16, 16)]
            ea = ea_v[pl.ds(g * 16, 16)]
            sk, sv = plsc.sort_key_val(di, ea)
            cs = plsc.cumsum(sv)
            k16_v[...] = sk
            c16_v[...] = cs
            v16_v[...] = sv
            prevk = plsc.load_gather(k16_v, [jnp.maximum(ii - 1, 0)])
            nextk = plsc.load_gather(k16_v, [jnp.minimum(ii + 1, 15)])
            isfirst = (ii == 0) | (sk != prevk)
            islast = (ii == 15) | (sk != nextk)
            f = plsc.cummax(jnp.where(isfirst, ii, 0))
            cf = plsc.load_gather(c16_v, [f])
            svf = plsc.load_gather(v16_v, [f])
            tot = cs - cf + svf
            plsc.addupdate_scatter(den_v, [sk], tot, mask=islast)

    def _issue_idx(sb, pb):
        c0 = wid * TPC + sb * SB
        pltpu.async_copy(pk_hbm.at[pl.ds(c0, SB)], pk_v.at[pb], isems[pb])
        pltpu.async_copy(dst_hbm.at[pl.ds(c0, SB)], dstb_v.at[pb], isems[pb])

    def _drain_idx(pb):
        pltpu.make_async_copy(pk_hbm.at[pl.ds(0, SB)], pk_v.at[pb],
                              isems[pb]).wait()
        pltpu.make_async_copy(dst_hbm.at[pl.ds(0, SB)], dstb_v.at[pb],
                              isems[pb]).wait()

    _issue_idx(0, 0)

    def _run_sb(sb, pb):
        _drain_idx(pb)

        @pl.when(sb + 1 < NSB)
        def _prefetch():
            _issue_idx(sb + 1, 1 - pb)

        descs = [None, None]
        descs[0] = pltpu.async_copy(
            h_hbm.at[pk_v.at[pb, 0, pl.ds(0, C)]], rows_v.at[0], sems[0]
        )
        for jj in range(SB):
            bb = jj % 2
            descs[bb].wait()
            if jj + 1 < SB:
                nb = (jj + 1) % 2
                descs[nb] = pltpu.async_copy(
                    h_hbm.at[pk_v.at[pb, jj + 1, pl.ds(0, C)]],
                    rows_v.at[nb], sems[nb]
                )
            _compute(jj, bb, pb)
            sdesc = pltpu.async_copy(
                rows_v.at[bb], acc_sh.at[dstb_v.at[pb, jj]], ssems[bb],
                add=True)
            _denom(jj, pb)
            sdesc.wait()

    def _pair(i, _):
        _run_sb(2 * i, 0)
        _run_sb(2 * i + 1, 1)
        return 0

    lax.fori_loop(0, NSB // 2, _pair, 0)

    plsc.subcore_barrier()
    pltpu.sync_copy(
        acc_sh.at[pl.ds(sid * ROWS_PER_SUB, ROWS_PER_SUB)],
        out_hbm.at[cid, pl.ds(sid * ROWS_PER_SUB, ROWS_PER_SUB)],
    )
    pltpu.sync_copy(den_v, den_hbm.at[wid])


_sc_mesh = plsc.VectorSubcoreMesh(
    core_axis_name="c", subcore_axis_name="s", num_cores=NCORE,
    num_subcores=NSUB,
)

_sc_edge_pass = functools.partial(
    pl.kernel,
    mesh=_sc_mesh,
    out_type=[
        jax.ShapeDtypeStruct((NCORE, NPAD, D), jnp.float32),
        jax.ShapeDtypeStruct((NW, N), jnp.float32),
    ],
    compiler_params=pltpu.CompilerParams(needs_layout_passes=False),
    scratch_types=[
        pltpu.VMEM((N,), jnp.float32),
        pltpu.VMEM((N,), jnp.float32),
        pltpu.VMEM((2, SB, 2 * C), jnp.int32),
        pltpu.VMEM((2, SB, C), jnp.int32),
        pltpu.VMEM((C,), jnp.float32),
        pltpu.VMEM((2, C, D), jnp.float32),
        pltpu.VMEM((N,), jnp.float32),
        pltpu.VMEM((16,), jnp.int32),
        pltpu.VMEM((16,), jnp.float32),
        pltpu.VMEM((16,), jnp.float32),
        pltpu.VMEM_SHARED((NPAD, D), jnp.float32),
        pltpu.SemaphoreType.DMA,
        pltpu.SemaphoreType.DMA,
        pltpu.SemaphoreType.DMA,
        pltpu.SemaphoreType.DMA,
        pltpu.SemaphoreType.DMA,
        pltpu.SemaphoreType.DMA,
    ],
)(_sc_edge_body)


def kernel(x, edge_index, batch, edge_attr, W1, We1, asrc1, adst1, ae1, b1,
           g1, be1, W2, We2, asrc2, adst2, ae2, b2, g2, be2, Wl, bl, gl, bel):
    npadedge = (NCHUNK_P - NCHUNK) * C
    pidx = jnp.arange(npadedge, dtype=jnp.int32)
    psrc = (pidx % N).reshape(NCHUNK_P - NCHUNK, C)
    pdst = (pidx % N).reshape(NCHUNK_P - NCHUNK, C)
    src = jnp.concatenate(
        [edge_index[0].reshape(NCHUNK, C), psrc], axis=0)
    dst = jnp.concatenate(
        [edge_index[1].reshape(NCHUNK, C), pdst], axis=0)
    et1, et2 = _eterms(edge_attr.T, We1, ae1, We2, ae2)
    pk1 = jnp.concatenate(
        [src, lax.bitcast_convert_type(et1, jnp.int32)], axis=1)
    pk2 = jnp.concatenate(
        [src, lax.bitcast_convert_type(et2, jnp.int32)], axis=1)

    h1, s1, d1 = _prep(x, W1, asrc1, adst1)
    acc1, den1 = _sc_edge_pass(h1, s1, d1, pk1, dst)

    h2, s2, d2 = _mid(acc1, den1, b1, g1, be1, W2, asrc2, adst2)
    acc2, den2 = _sc_edge_pass(h2, s2, d2, pk2, dst)

    return _final(acc2, den2, b2, g2, be2, batch.reshape(N, 1), Wl, bl, gl,
                  bel)


# speedup vs baseline: 1.0266x; 1.0266x over previous
import functools

import jax
import jax.numpy as jnp
from jax import lax
from jax.experimental import pallas as pl
from jax.experimental.pallas import tpu as pltpu
from jax.experimental.pallas import tpu_sc as plsc

N = 10000
E = 320000
D = 128
ED = 16
B = 64
NCORE = 2
NSUB = 16
NW = NCORE * NSUB
C = 64
NCHUNK = E // C
NCHUNK_P = 5120
TPC = NCHUNK_P // NW
SB = 4
NSB = TPC // SB
NPAD = 10112
ROWS_PER_SUB = NPAD // NSUB


def _eterm_body(eat_ref, we1_ref, ae1_ref, we2_ref, ae2_ref, o1_ref, o2_ref):
    w1 = jnp.dot(we1_ref[...], ae1_ref[...], preferred_element_type=jnp.float32)
    w2 = jnp.dot(we2_ref[...], ae2_ref[...], preferred_element_type=jnp.float32)
    a = eat_ref[...]
    m1 = lax.dot_general(
        w1, a, (((0,), (0,)), ((), ())), preferred_element_type=jnp.float32
    )
    m2 = lax.dot_general(
        w2, a, (((0,), (0,)), ((), ())), preferred_element_type=jnp.float32
    )
    blk = m1.shape[1]
    gidx = lax.broadcasted_iota(jnp.int32, (1, blk), 1) + pl.program_id(0) * blk
    m1 = jnp.where(gidx < E, m1, -1e30)
    m2 = jnp.where(gidx < E, m2, -1e30)
    o1_ref[...] = m1[0]
    o2_ref[...] = m2[0]


def _eterms(edge_attr_t, We1, ae1, We2, ae2):
    blk = 32768
    EP = NCHUNK_P * C
    grid = EP // blk
    out = pl.pallas_call(
        _eterm_body,
        grid=(grid,),
        in_specs=[
            pl.BlockSpec((ED, blk), lambda i: (0, i)),
            pl.BlockSpec((ED, D), lambda i: (0, 0)),
            pl.BlockSpec((D, 1), lambda i: (0, 0)),
            pl.BlockSpec((ED, D), lambda i: (0, 0)),
            pl.BlockSpec((D, 1), lambda i: (0, 0)),
        ],
        out_specs=[
            pl.BlockSpec((blk,), lambda i: (i,)),
            pl.BlockSpec((blk,), lambda i: (i,)),
        ],
        out_shape=[
            jax.ShapeDtypeStruct((EP,), jnp.float32),
            jax.ShapeDtypeStruct((EP,), jnp.float32),
        ],
    )(edge_attr_t, We1, ae1.reshape(D, 1), We2, ae2.reshape(D, 1))
    return out[0].reshape(NCHUNK_P, C), out[1].reshape(NCHUNK_P, C)


def _prep_body(x_ref, w_ref, asrc_ref, adst_ref, h_ref, s_ref, d_ref):
    h = jnp.dot(x_ref[...], w_ref[...], preferred_element_type=jnp.float32)
    h_ref[...] = h
    s_ref[...] = jnp.sum(h * asrc_ref[...], axis=1)
    d_ref[...] = jnp.sum(h * adst_ref[...], axis=1)


def _prep(xin, W, asrc, adst):
    return pl.pallas_call(
        _prep_body,
        out_shape=[
            jax.ShapeDtypeStruct((N, D), jnp.float32),
            jax.ShapeDtypeStruct((N,), jnp.float32),
            jax.ShapeDtypeStruct((N,), jnp.float32),
        ],
    )(xin, W, asrc.reshape(1, D), adst.reshape(1, D))


def _layer_norm(v, g, b):
    mu = jnp.mean(v, axis=-1, keepdims=True)
    var = jnp.mean((v - mu) ** 2, axis=-1, keepdims=True)
    return (v - mu) / jnp.sqrt(var + 1e-5) * g + b


def _combine(acc_ref, denp_ref, b_ref):
    asum = acc_ref[0] + acc_ref[1]
    u = asum[: N, :]
    ones = jnp.ones((NW, 1), jnp.float32)
    den = lax.dot_general(
        denp_ref[...], ones, (((0,), (0,)), ((), ())),
        preferred_element_type=jnp.float32,
    )
    return u / (den[: N] + 1e-16) + b_ref[...]


def _mid_body(acc_ref, denp_ref, b_ref, g_ref, be_ref, w_ref, asrc_ref,
              adst_ref, h_ref, s_ref, d_ref):
    x1 = _combine(acc_ref, denp_ref, b_ref)
    x2 = _layer_norm(x1, g_ref[...], be_ref[...])
    h = jnp.dot(x2, w_ref[...], preferred_element_type=jnp.float32)
    h_ref[...] = h
    s_ref[...] = jnp.sum(h * asrc_ref[...], axis=1)
    d_ref[...] = jnp.sum(h * adst_ref[...], axis=1)


def _mid(acc, denp, b1, g1, be1, W2, asrc2, adst2):
    return pl.pallas_call(
        _mid_body,
        out_shape=[
            jax.ShapeDtypeStruct((N, D), jnp.float32),
            jax.ShapeDtypeStruct((N,), jnp.float32),
            jax.ShapeDtypeStruct((N,), jnp.float32),
        ],
    )(acc, denp, b1.reshape(1, D), g1.reshape(1, D), be1.reshape(1, D), W2,
      asrc2.reshape(1, D), adst2.reshape(1, D))


def _final_body(acc_ref, denp_ref, b_ref, g_ref, be_ref, batch_ref, wl_ref,
                bl_ref, gl_ref, bel_ref, out_ref):
    x3 = _combine(acc_ref, denp_ref, b_ref)
    x4 = _layer_norm(x3, g_ref[...], be_ref[...])
    oh = jnp.where(
        batch_ref[...] == lax.broadcasted_iota(jnp.int32, (N, B), 1), 1.0, 0.0
    )
    sums = lax.dot_general(
        oh, x4, (((0,), (0,)), ((), ())), preferred_element_type=jnp.float32
    )
    counts = lax.dot_general(
        oh, jnp.ones((N, 1), jnp.float32), (((0,), (0,)), ((), ())),
        preferred_element_type=jnp.float32,
    )
    pooled = sums / jnp.maximum(counts, 1.0)
    x5 = jnp.dot(pooled, wl_ref[...], preferred_element_type=jnp.float32)
    x5 = x5 + bl_ref[...]
    x6 = _layer_norm(x5, gl_ref[...], bel_ref[...])
    out_ref[...] = jnp.maximum(x6, 0.0)


def _final(acc, denp, b2, g2, be2, batch2d, Wl, bl, gl, bel):
    return pl.pallas_call(
        _final_body,
        out_shape=jax.ShapeDtypeStruct((B, D), jnp.float32),
    )(acc, denp, b2.reshape(1, D), g2.reshape(1, D), be2.reshape(1, D),
      batch2d, Wl, bl.reshape(1, D), gl.reshape(1, D), bel.reshape(1, D))


def _sc_edge_body(h_hbm, s_hbm, d_hbm, pk_hbm, dst_hbm,
                  out_hbm, den_hbm,
                  s_v, d_v, pk_v, dstb_v, ea_v, rows_v, den_v,
                  k16_v, c16_v, v16_v, acc_sh, sem0, sem1, isem0, isem1,
                  ssem0, ssem1):
    cid = lax.axis_index("c")
    sid = lax.axis_index("s")
    wid = sid * NCORE + cid
    ii = lax.iota(jnp.int32, 16)
    sems = (sem0, sem1)
    isems = (isem0, isem1)
    ssems = (ssem0, ssem1)

    def _zrow(i, _):
        for k in range(D // 16):
            rows_v[0, i, pl.ds(k * 16, 16)] = jnp.zeros((16,), jnp.float32)
        return 0

    lax.fori_loop(0, C, _zrow, 0)

    def _zden(i, _):
        den_v[pl.ds(i * 16, 16)] = jnp.zeros((16,), jnp.float32)
        return 0

    lax.fori_loop(0, N // 16, _zden, 0)
    for j in range(ROWS_PER_SUB // C):
        pltpu.sync_copy(
            rows_v.at[0],
            acc_sh.at[pl.ds(sid * ROWS_PER_SUB + j * C, C)],
        )
    _rem = ROWS_PER_SUB % C
    if _rem:
        pltpu.sync_copy(
            rows_v.at[0, pl.ds(0, _rem)],
            acc_sh.at[pl.ds(sid * ROWS_PER_SUB + (ROWS_PER_SUB // C) * C,
                            _rem)],
        )

    pltpu.sync_copy(s_hbm, s_v)
    pltpu.sync_copy(d_hbm, d_v)
    plsc.subcore_barrier()

    def _compute(jj, bb, pb):
        @plsc.parallel_loop(0, C // 16, unroll=C // 16)
        def _alpha(g):
            si = pk_v[pb, jj, pl.ds(g * 16, 16)]
            di = dstb_v[pb, jj, pl.ds(g * 16, 16)]
            et = plsc.bitcast(pk_v[pb, jj, pl.ds(C + g * 16, 16)],
                              jnp.float32)
            a = plsc.load_gather(s_v, [si]) + plsc.load_gather(d_v, [di]) + et
            a = jnp.maximum(a, 0.2 * a)
            ea = jnp.exp(a)
            ea_v[pl.ds(g * 16, 16)] = ea

        @plsc.parallel_loop(0, C, unroll=4)
        def _scale(e):
            bc = plsc.load_gather(ea_v, [jnp.full((16,), e, jnp.int32)])
            for k in range(D // 16):
                rows_v[bb, e, pl.ds(k * 16, 16)] = (
                    rows_v[bb, e, pl.ds(k * 16, 16)] * bc
                )

    def _denom(jj, pb):
        for g in range(C // 16):
            di = dstb_v[pb, jj, pl.ds(g * 16, 16)]
            ea = ea_v[pl.ds(g * 16, 16)]
            sk, sv = plsc.sort_key_val(di, ea)
            cs = plsc.cumsum(sv)
            k16_v[...] = sk
            c16_v[...] = cs
            v16_v[...] = sv
            prevk = plsc.load_gather(k16_v, [jnp.maximum(ii - 1, 0)])
            nextk = plsc.load_gather(k16_v, [jnp.minimum(ii + 1, 15)])
            isfirst = (ii == 0) | (sk != prevk)
            islast = (ii == 15) | (sk != nextk)
            f = plsc.cummax(jnp.where(isfirst, ii, 0))
            cf = plsc.load_gather(c16_v, [f])
            svf = plsc.load_gather(v16_v, [f])
            tot = cs - cf + svf
            plsc.addupdate_scatter(den_v, [sk], tot, mask=islast)

    def _issue_idx(sb, pb):
        c0 = wid * TPC + sb * SB
        pltpu.async_copy(pk_hbm.at[pl.ds(c0, SB)], pk_v.at[pb], isems[pb])
        pltpu.async_copy(dst_hbm.at[pl.ds(c0, SB)], dstb_v.at[pb], isems[pb])

    def _drain_idx(pb):
        pltpu.make_async_copy(pk_hbm.at[pl.ds(0, SB)], pk_v.at[pb],
                              isems[pb]).wait()
        pltpu.make_async_copy(dst_hbm.at[pl.ds(0, SB)], dstb_v.at[pb],
                              isems[pb]).wait()

    _issue_idx(0, 0)

    def _run_sb(sb, pb):
        _drain_idx(pb)

        @pl.when(sb + 1 < NSB)
        def _prefetch():
            _issue_idx(sb + 1, 1 - pb)

        descs = [None, None]
        descs[0] = pltpu.async_copy(
            h_hbm.at[pk_v.at[pb, 0, pl.ds(0, C)]], rows_v.at[0], sems[0]
        )
        for jj in range(SB):
            bb = jj % 2
            descs[bb].wait()
            if jj + 1 < SB:
                nb = (jj + 1) % 2
                descs[nb] = pltpu.async_copy(
                    h_hbm.at[pk_v.at[pb, jj + 1, pl.ds(0, C)]],
                    rows_v.at[nb], sems[nb]
                )
            _compute(jj, bb, pb)
            sdesc = pltpu.async_copy(
                rows_v.at[bb], acc_sh.at[dstb_v.at[pb, jj]], ssems[bb],
                add=True)
            _denom(jj, pb)
            sdesc.wait()

    def _pair(i, _):
        _run_sb(2 * i, 0)
        _run_sb(2 * i + 1, 1)
        return 0

    lax.fori_loop(0, NSB // 2, _pair, 0)

    plsc.subcore_barrier()
    pltpu.sync_copy(
        acc_sh.at[pl.ds(sid * ROWS_PER_SUB, ROWS_PER_SUB)],
        out_hbm.at[cid, pl.ds(sid * ROWS_PER_SUB, ROWS_PER_SUB)],
    )
    pltpu.sync_copy(den_v, den_hbm.at[wid])


_sc_mesh = plsc.VectorSubcoreMesh(
    core_axis_name="c", subcore_axis_name="s", num_cores=NCORE,
    num_subcores=NSUB,
)

_sc_edge_pass = functools.partial(
    pl.kernel,
    mesh=_sc_mesh,
    out_type=[
        jax.ShapeDtypeStruct((NCORE, NPAD, D), jnp.float32),
        jax.ShapeDtypeStruct((NW, N), jnp.float32),
    ],
    compiler_params=pltpu.CompilerParams(needs_layout_passes=False),
    scratch_types=[
        pltpu.VMEM((N,), jnp.float32),
        pltpu.VMEM((N,), jnp.float32),
        pltpu.VMEM((2, SB, 2 * C), jnp.int32),
        pltpu.VMEM((2, SB, C), jnp.int32),
        pltpu.VMEM((C,), jnp.float32),
        pltpu.VMEM((2, C, D), jnp.float32),
        pltpu.VMEM((N,), jnp.float32),
        pltpu.VMEM((16,), jnp.int32),
        pltpu.VMEM((16,), jnp.float32),
        pltpu.VMEM((16,), jnp.float32),
        pltpu.VMEM_SHARED((NPAD, D), jnp.float32),
        pltpu.SemaphoreType.DMA,
        pltpu.SemaphoreType.DMA,
        pltpu.SemaphoreType.DMA,
        pltpu.SemaphoreType.DMA,
        pltpu.SemaphoreType.DMA,
        pltpu.SemaphoreType.DMA,
    ],
)(_sc_edge_body)


def kernel(x, edge_index, batch, edge_attr, W1, We1, asrc1, adst1, ae1, b1,
           g1, be1, W2, We2, asrc2, adst2, ae2, b2, g2, be2, Wl, bl, gl, bel):
    npadedge = (NCHUNK_P - NCHUNK) * C
    pidx = jnp.arange(npadedge, dtype=jnp.int32)
    psrc = (pidx % N).reshape(NCHUNK_P - NCHUNK, C)
    pdst = (pidx % N).reshape(NCHUNK_P - NCHUNK, C)
    src = jnp.concatenate(
        [edge_index[0].reshape(NCHUNK, C), psrc], axis=0)
    dst = jnp.concatenate(
        [edge_index[1].reshape(NCHUNK, C), pdst], axis=0)
    et1, et2 = _eterms(edge_attr.T, We1, ae1, We2, ae2)
    pk1 = jnp.concatenate(
        [src, lax.bitcast_convert_type(et1, jnp.int32)], axis=1)
    pk2 = jnp.concatenate(
        [src, lax.bitcast_convert_type(et2, jnp.int32)], axis=1)

    h1, s1, d1 = _prep(x, W1, asrc1, adst1)
    acc1, den1 = _sc_edge_pass(h1, s1, d1, pk1, dst)

    h2, s2, d2 = _mid(acc1, den1, b1, g1, be1, W2, asrc2, adst2)
    acc2, den2 = _sc_edge_pass(h2, s2, d2, pk2, dst)

    return _final(acc2, den2, b2, g2, be2, batch.reshape(N, 1), Wl, bl, gl,
                  bel)


# speedup vs baseline: 1.0384x; 1.0114x over previous
import functools

import jax
import jax.numpy as jnp
from jax import lax
from jax.experimental import pallas as pl
from jax.experimental.pallas import tpu as pltpu
from jax.experimental.pallas import tpu_sc as plsc

N = 10000
E = 320000
D = 128
ED = 16
B = 64
NCORE = 2
NSUB = 16
NW = NCORE * NSUB
C = 64
NCHUNK = E // C
NCHUNK_P = 5120
TPC = NCHUNK_P // NW
SB = 4
NSB = TPC // SB
NPAD = 10112
ROWS_PER_SUB = NPAD // NSUB


def _eterm_body(eat_ref, we1_ref, ae1_ref, we2_ref, ae2_ref, o1_ref, o2_ref):
    w1 = jnp.dot(we1_ref[...], ae1_ref[...], preferred_element_type=jnp.float32)
    w2 = jnp.dot(we2_ref[...], ae2_ref[...], preferred_element_type=jnp.float32)
    a = eat_ref[...]
    m1 = lax.dot_general(
        w1, a, (((0,), (0,)), ((), ())), preferred_element_type=jnp.float32
    )
    m2 = lax.dot_general(
        w2, a, (((0,), (0,)), ((), ())), preferred_element_type=jnp.float32
    )
    blk = m1.shape[1]
    gidx = lax.broadcasted_iota(jnp.int32, (1, blk), 1) + pl.program_id(0) * blk
    m1 = jnp.where(gidx < E, m1, -1e30)
    m2 = jnp.where(gidx < E, m2, -1e30)
    o1_ref[...] = m1[0]
    o2_ref[...] = m2[0]


def _eterms(edge_attr_t, We1, ae1, We2, ae2):
    blk = 32768
    EP = NCHUNK_P * C
    grid = EP // blk
    out = pl.pallas_call(
        _eterm_body,
        grid=(grid,),
        in_specs=[
            pl.BlockSpec((ED, blk), lambda i: (0, i)),
            pl.BlockSpec((ED, D), lambda i: (0, 0)),
            pl.BlockSpec((D, 1), lambda i: (0, 0)),
            pl.BlockSpec((ED, D), lambda i: (0, 0)),
            pl.BlockSpec((D, 1), lambda i: (0, 0)),
        ],
        out_specs=[
            pl.BlockSpec((blk,), lambda i: (i,)),
            pl.BlockSpec((blk,), lambda i: (i,)),
        ],
        out_shape=[
            jax.ShapeDtypeStruct((EP,), jnp.float32),
            jax.ShapeDtypeStruct((EP,), jnp.float32),
        ],
    )(edge_attr_t, We1, ae1.reshape(D, 1), We2, ae2.reshape(D, 1))
    return out[0].reshape(NCHUNK_P, C), out[1].reshape(NCHUNK_P, C)


def _prep_body(x_ref, w_ref, asrc_ref, adst_ref, h_ref, s_ref, d_ref):
    h = jnp.dot(x_ref[...], w_ref[...], preferred_element_type=jnp.float32)
    h_ref[...] = h
    s_ref[...] = jnp.sum(h * asrc_ref[...], axis=1)
    d_ref[...] = jnp.sum(h * adst_ref[...], axis=1)


def _prep(xin, W, asrc, adst):
    return pl.pallas_call(
        _prep_body,
        out_shape=[
            jax.ShapeDtypeStruct((N, D), jnp.float32),
            jax.ShapeDtypeStruct((N,), jnp.float32),
            jax.ShapeDtypeStruct((N,), jnp.float32),
        ],
    )(xin, W, asrc.reshape(1, D), adst.reshape(1, D))


def _layer_norm(v, g, b):
    mu = jnp.mean(v, axis=-1, keepdims=True)
    var = jnp.mean((v - mu) ** 2, axis=-1, keepdims=True)
    return (v - mu) / jnp.sqrt(var + 1e-5) * g + b


def _combine(acc_ref, denp_ref, b_ref):
    asum = acc_ref[0] + acc_ref[1]
    u = asum[: N, :]
    ones = jnp.ones((NW, 1), jnp.float32)
    den = lax.dot_general(
        denp_ref[...], ones, (((0,), (0,)), ((), ())),
        preferred_element_type=jnp.float32,
    )
    return u / (den[: N] + 1e-16) + b_ref[...]


def _mid_body(acc_ref, denp_ref, b_ref, g_ref, be_ref, w_ref, asrc_ref,
              adst_ref, h_ref, s_ref, d_ref):
    x1 = _combine(acc_ref, denp_ref, b_ref)
    x2 = _layer_norm(x1, g_ref[...], be_ref[...])
    h = jnp.dot(x2, w_ref[...], preferred_element_type=jnp.float32)
    h_ref[...] = h
    s_ref[...] = jnp.sum(h * asrc_ref[...], axis=1)
    d_ref[...] = jnp.sum(h * adst_ref[...], axis=1)


def _mid(acc, denp, b1, g1, be1, W2, asrc2, adst2):
    return pl.pallas_call(
        _mid_body,
        out_shape=[
            jax.ShapeDtypeStruct((N, D), jnp.float32),
            jax.ShapeDtypeStruct((N,), jnp.float32),
            jax.ShapeDtypeStruct((N,), jnp.float32),
        ],
    )(acc, denp, b1.reshape(1, D), g1.reshape(1, D), be1.reshape(1, D), W2,
      asrc2.reshape(1, D), adst2.reshape(1, D))


def _final_body(acc_ref, denp_ref, b_ref, g_ref, be_ref, batch_ref, wl_ref,
                bl_ref, gl_ref, bel_ref, out_ref):
    x3 = _combine(acc_ref, denp_ref, b_ref)
    x4 = _layer_norm(x3, g_ref[...], be_ref[...])
    oh = jnp.where(
        batch_ref[...] == lax.broadcasted_iota(jnp.int32, (N, B), 1), 1.0, 0.0
    )
    sums = lax.dot_general(
        oh, x4, (((0,), (0,)), ((), ())), preferred_element_type=jnp.float32
    )
    counts = lax.dot_general(
        oh, jnp.ones((N, 1), jnp.float32), (((0,), (0,)), ((), ())),
        preferred_element_type=jnp.float32,
    )
    pooled = sums / jnp.maximum(counts, 1.0)
    x5 = jnp.dot(pooled, wl_ref[...], preferred_element_type=jnp.float32)
    x5 = x5 + bl_ref[...]
    x6 = _layer_norm(x5, gl_ref[...], bel_ref[...])
    out_ref[...] = jnp.maximum(x6, 0.0)


def _final(acc, denp, b2, g2, be2, batch2d, Wl, bl, gl, bel):
    return pl.pallas_call(
        _final_body,
        out_shape=jax.ShapeDtypeStruct((B, D), jnp.float32),
    )(acc, denp, b2.reshape(1, D), g2.reshape(1, D), be2.reshape(1, D),
      batch2d, Wl, bl.reshape(1, D), gl.reshape(1, D), bel.reshape(1, D))


def _sc_edge_body(h_hbm, s_hbm, d_hbm, pk_hbm, dst_hbm,
                  out_hbm, den_hbm,
                  s_v, d_v, pk_v, dstb_v, ea_v, rows_v, den_v,
                  k16_v, c16_v, v16_v, acc_sh, sem0, sem1, isem0, isem1,
                  ssem0, ssem1):
    cid = lax.axis_index("c")
    sid = lax.axis_index("s")
    wid = sid * NCORE + cid
    ii = lax.iota(jnp.int32, 16)
    sems = (sem0, sem1)
    isems = (isem0, isem1)
    ssems = (ssem0, ssem1)

    def _zrow(i, _):
        for k in range(D // 16):
            rows_v[0, i, pl.ds(k * 16, 16)] = jnp.zeros((16,), jnp.float32)
        return 0

    lax.fori_loop(0, C, _zrow, 0)

    def _zden(i, _):
        den_v[pl.ds(i * 16, 16)] = jnp.zeros((16,), jnp.float32)
        return 0

    lax.fori_loop(0, N // 16, _zden, 0)
    zdescs = []
    for j in range(ROWS_PER_SUB // C):
        zdescs.append(pltpu.async_copy(
            rows_v.at[0],
            acc_sh.at[pl.ds(sid * ROWS_PER_SUB + j * C, C)], sem0,
        ))
    _rem = ROWS_PER_SUB % C
    if _rem:
        zdescs.append(pltpu.async_copy(
            rows_v.at[0, pl.ds(0, _rem)],
            acc_sh.at[pl.ds(sid * ROWS_PER_SUB + (ROWS_PER_SUB // C) * C,
                            _rem)], sem0,
        ))

    sdesc0 = pltpu.async_copy(s_hbm, s_v, sem1)
    ddesc0 = pltpu.async_copy(d_hbm, d_v, ssem0)
    for zd in zdescs:
        zd.wait()
    sdesc0.wait()
    ddesc0.wait()
    plsc.subcore_barrier()

    def _compute(jj, bb, pb):
        @plsc.parallel_loop(0, C // 16, unroll=C // 16)
        def _alpha(g):
            si = pk_v[pb, jj, pl.ds(g * 16, 16)]
            di = dstb_v[pb, jj, pl.ds(g * 16, 16)]
            et = plsc.bitcast(pk_v[pb, jj, pl.ds(C + g * 16, 16)],
                              jnp.float32)
            a = plsc.load_gather(s_v, [si]) + plsc.load_gather(d_v, [di]) + et
            a = jnp.maximum(a, 0.2 * a)
            ea = jnp.exp(a)
            ea_v[pl.ds(g * 16, 16)] = ea

        @plsc.parallel_loop(0, C, unroll=4)
        def _scale(e):
            bc = plsc.load_gather(ea_v, [jnp.full((16,), e, jnp.int32)])
            for k in range(D // 16):
                rows_v[bb, e, pl.ds(k * 16, 16)] = (
                    rows_v[bb, e, pl.ds(k * 16, 16)] * bc
                )

    def _denom(jj, pb):
        for g in range(C // 16):
            di = dstb_v[pb, jj, pl.ds(g * 16, 16)]
            ea = ea_v[pl.ds(g * 16, 16)]
            sk, sv = plsc.sort_key_val(di, ea)
            cs = plsc.cumsum(sv)
            k16_v[...] = sk
            c16_v[...] = cs
            v16_v[...] = sv
            prevk = plsc.load_gather(k16_v, [jnp.maximum(ii - 1, 0)])
            nextk = plsc.load_gather(k16_v, [jnp.minimum(ii + 1, 15)])
            isfirst = (ii == 0) | (sk != prevk)
            islast = (ii == 15) | (sk != nextk)
            f = plsc.cummax(jnp.where(isfirst, ii, 0))
            cf = plsc.load_gather(c16_v, [f])
            svf = plsc.load_gather(v16_v, [f])
            tot = cs - cf + svf
            plsc.addupdate_scatter(den_v, [sk], tot, mask=islast)

    def _issue_idx(sb, pb):
        c0 = wid * TPC + sb * SB
        pltpu.async_copy(pk_hbm.at[pl.ds(c0, SB)], pk_v.at[pb], isems[pb])
        pltpu.async_copy(dst_hbm.at[pl.ds(c0, SB)], dstb_v.at[pb], isems[pb])

    def _drain_idx(pb):
        pltpu.make_async_copy(pk_hbm.at[pl.ds(0, SB)], pk_v.at[pb],
                              isems[pb]).wait()
        pltpu.make_async_copy(dst_hbm.at[pl.ds(0, SB)], dstb_v.at[pb],
                              isems[pb]).wait()

    _issue_idx(0, 0)

    def _run_sb(sb, pb):
        _drain_idx(pb)

        @pl.when(sb + 1 < NSB)
        def _prefetch():
            _issue_idx(sb + 1, 1 - pb)

        descs = [None, None]
        descs[0] = pltpu.async_copy(
            h_hbm.at[pk_v.at[pb, 0, pl.ds(0, C)]], rows_v.at[0], sems[0]
        )
        for jj in range(SB):
            bb = jj % 2
            descs[bb].wait()
            if jj + 1 < SB:
                nb = (jj + 1) % 2
                descs[nb] = pltpu.async_copy(
                    h_hbm.at[pk_v.at[pb, jj + 1, pl.ds(0, C)]],
                    rows_v.at[nb], sems[nb]
                )
            _compute(jj, bb, pb)
            sdesc = pltpu.async_copy(
                rows_v.at[bb], acc_sh.at[dstb_v.at[pb, jj]], ssems[bb],
                add=True)
            _denom(jj, pb)
            sdesc.wait()

    def _pair(i, _):
        _run_sb(2 * i, 0)
        _run_sb(2 * i + 1, 1)
        return 0

    lax.fori_loop(0, NSB // 2, _pair, 0)

    plsc.subcore_barrier()
    pltpu.sync_copy(
        acc_sh.at[pl.ds(sid * ROWS_PER_SUB, ROWS_PER_SUB)],
        out_hbm.at[cid, pl.ds(sid * ROWS_PER_SUB, ROWS_PER_SUB)],
    )
    pltpu.sync_copy(den_v, den_hbm.at[wid])


_sc_mesh = plsc.VectorSubcoreMesh(
    core_axis_name="c", subcore_axis_name="s", num_cores=NCORE,
    num_subcores=NSUB,
)

_sc_edge_pass = functools.partial(
    pl.kernel,
    mesh=_sc_mesh,
    out_type=[
        jax.ShapeDtypeStruct((NCORE, NPAD, D), jnp.float32),
        jax.ShapeDtypeStruct((NW, N), jnp.float32),
    ],
    compiler_params=pltpu.CompilerParams(needs_layout_passes=False),
    scratch_types=[
        pltpu.VMEM((N,), jnp.float32),
        pltpu.VMEM((N,), jnp.float32),
        pltpu.VMEM((2, SB, 2 * C), jnp.int32),
        pltpu.VMEM((2, SB, C), jnp.int32),
        pltpu.VMEM((C,), jnp.float32),
        pltpu.VMEM((2, C, D), jnp.float32),
        pltpu.VMEM((N,), jnp.float32),
        pltpu.VMEM((16,), jnp.int32),
        pltpu.VMEM((16,), jnp.float32),
        pltpu.VMEM((16,), jnp.float32),
        pltpu.VMEM_SHARED((NPAD, D), jnp.float32),
        pltpu.SemaphoreType.DMA,
        pltpu.SemaphoreType.DMA,
        pltpu.SemaphoreType.DMA,
        pltpu.SemaphoreType.DMA,
        pltpu.SemaphoreType.DMA,
        pltpu.SemaphoreType.DMA,
    ],
)(_sc_edge_body)


def kernel(x, edge_index, batch, edge_attr, W1, We1, asrc1, adst1, ae1, b1,
           g1, be1, W2, We2, asrc2, adst2, ae2, b2, g2, be2, Wl, bl, gl, bel):
    npadedge = (NCHUNK_P - NCHUNK) * C
    pidx = jnp.arange(npadedge, dtype=jnp.int32)
    psrc = (pidx % N).reshape(NCHUNK_P - NCHUNK, C)
    pdst = (pidx % N).reshape(NCHUNK_P - NCHUNK, C)
    src = jnp.concatenate(
        [edge_index[0].reshape(NCHUNK, C), psrc], axis=0)
    dst = jnp.concatenate(
        [edge_index[1].reshape(NCHUNK, C), pdst], axis=0)
    et1, et2 = _eterms(edge_attr.T, We1, ae1, We2, ae2)
    pk1 = jnp.concatenate(
        [src, lax.bitcast_convert_type(et1, jnp.int32)], axis=1)
    pk2 = jnp.concatenate(
        [src, lax.bitcast_convert_type(et2, jnp.int32)], axis=1)

    h1, s1, d1 = _prep(x, W1, asrc1, adst1)
    acc1, den1 = _sc_edge_pass(h1, s1, d1, pk1, dst)

    h2, s2, d2 = _mid(acc1, den1, b1, g1, be1, W2, asrc2, adst2)
    acc2, den2 = _sc_edge_pass(h2, s2, d2, pk2, dst)

    return _final(acc2, den2, b2, g2, be2, batch.reshape(N, 1), Wl, bl, gl,
                  bel)
